# baseline (speedup 1.0000x reference)
import functools

import jax
import jax.numpy as jnp
from jax import lax
from jax.experimental import pallas as pl
from jax.experimental.pallas import tpu as pltpu
from jax.experimental.pallas import tpu_sc as plsc

_NUM_CORES = 2
_NUM_SUBCORES = 16
_NUM_WORKERS = _NUM_CORES * _NUM_SUBCORES

_RPC = 4
_SPLITS = ((0, 104), (104, 96))


def _project_body(emb_ref, w_ref, b_ref, out_ref):
    out_ref[...] = lax.dot_general(
        emb_ref[...], w_ref[...],
        dimension_numbers=(((1,), (1,)), ((), ())),
        preferred_element_type=jnp.float32,
    ) + b_ref[...]


def _project_table(embed_table, proj_w, proj_b):
    v, d = embed_table.shape
    return pl.pallas_call(
        _project_body,
        out_shape=jax.ShapeDtypeStruct((v, d), jnp.float32),
    )(embed_table, proj_w, proj_b.reshape(1, d))


def _sc_lookup(table_proj, x2d):
    bsz, seq = x2d.shape
    v, d = table_proj.shape
    rows_per_worker = bsz // _NUM_WORKERS
    n_chunks = rows_per_worker // _RPC
    half = n_chunks // 2

    mesh = plsc.VectorSubcoreMesh(
        core_axis_name="c", subcore_axis_name="s")

    @functools.partial(
        pl.kernel,
        out_type=jax.ShapeDtypeStruct((bsz, seq, d), jnp.float32),
        mesh=mesh,
        scratch_types=[
            pltpu.VMEM_SHARED((v, d), jnp.float32),
            pltpu.VMEM((2, _RPC, seq), jnp.int32),
            pltpu.VMEM((2, _RPC, seq, d), jnp.float32),
            [pltpu.SemaphoreType.DMA] * 2,
            [pltpu.SemaphoreType.DMA] * 2,
            [pltpu.SemaphoreType.DMA] * 2,
        ],
        compiler_params=pltpu.CompilerParams(use_tc_tiling_on_sc=False),
    )
    def lookup(table_hbm, idx_hbm, out_hbm, table_sh, idx_v, rows_v,
               isem, gsem, osem):
        sid = lax.axis_index("s")
        wid = sid * _NUM_CORES + lax.axis_index("c")
        base = wid * rows_per_worker

        @pl.when(sid == 0)
        def _():
            pltpu.sync_copy(table_hbm, table_sh)
        plsc.subcore_barrier()

        def idx_copy(c, p):
            row = pl.multiple_of(base + c * _RPC, _RPC)
            return pltpu.make_async_copy(
                idx_hbm.at[pl.ds(row, _RPC)], idx_v.at[p], isem[p])

        def gathers(c, p):
            del c
            return [
                pltpu.make_async_copy(
                    table_sh.at[idx_v.at[p, r, pl.ds(o, g)]],
                    rows_v.at[p, r, pl.ds(o, g)],
                    gsem[p])
                for r in range(_RPC)
                for o, g in _SPLITS
            ]

        def write(c, p):
            row = pl.multiple_of(base + c * _RPC, _RPC)
            return pltpu.make_async_copy(
                rows_v.at[p], out_hbm.at[pl.ds(row, _RPC)], osem[p])

        idx_copy(0, 0).start()
        idx_copy(1, 1).start()

        def retire(c, q, prefetch_pred):
            for g in gathers(c, q):
                g.wait()
            write(c, q).start()
            if prefetch_pred is True:
                idx_copy(c + 2, q).start()
            else:
                @pl.when(prefetch_pred)
                def _():
                    idx_copy(c + 2, q).start()

        def body(g, carry):
            for u in (0, 1):
                c = 2 * g + u
                p, q = u, 1 - u
                @pl.when(g >= 1)
                def _():
                    write(c - 2, p).wait()
                idx_copy(c, p).wait()
                for gd in gathers(c, p):
                    gd.start()
                if u == 0:
                    @pl.when(g >= 1)
                    def _():
                        retire(c - 1, q, True)
                else:
                    retire(c - 1, q, c + 2 < n_chunks)
            return carry

        lax.fori_loop(0, half, body, 0)

        last = n_chunks - 1
        for g in gathers(last, 1):
            g.wait()
        write(last, 1).start()
        write(last - 1, 0).wait()
        write(last, 1).wait()

    return lookup(table_proj, x2d)


def kernel(x, embed_table, proj_w, proj_b):
    table_proj = _project_table(embed_table, proj_w, proj_b)
    return _sc_lookup(table_proj, x.astype(jnp.int32))
